# Initial kernel scaffold; baseline (speedup 1.0000x reference)
import functools

import jax
import jax.numpy as jnp
from jax import lax
from jax.experimental import pallas as pl
from jax.experimental.pallas import tpu as pltpu
from jax.experimental.pallas import tpu_sc as plsc

H = 16
S = 4096
D = 64
N_IMG = 3072
N_TXT = S - N_IMG
TOP_K = N_IMG // 2
N_REM = N_IMG - TOP_K
N_GPU = N_TXT + TOP_K
T_BLK = 256
N_TBLK = N_TXT // T_BLK

NW = 32
G_PER = H * N_GPU // NW
C_PER = H * N_REM // NW


def _scores_body(q_ref, k_ref, out_ref):
    q = q_ref[0]
    k = k_ref[0]
    logits = lax.dot_general(
        q, k, (((1,), (1,)), ((), ())),
        preferred_element_type=jnp.float32) * 0.125
    m = jnp.max(logits, axis=1, keepdims=True)
    e = jnp.exp(logits - m)
    z = jnp.sum(e, axis=1, keepdims=True)
    part = jnp.sum(e / z, axis=0)[None, None, :]

    @pl.when(pl.program_id(1) == 0)
    def _():
        out_ref[...] = part

    @pl.when(pl.program_id(1) != 0)
    def _():
        out_ref[...] += part


def _roll(x, shift, n):
    return pltpu.roll(x, shift % n, 1)


def _bitonic(keys, vals, n, cmp):
    pos = lax.broadcasted_iota(jnp.int32, keys.shape, 1)
    lvl = 2
    while lvl <= n:
        j = lvl // 2
        while j >= 1:
            upper = (pos & j) != 0
            pk = jnp.where(upper, _roll(keys, j, n), _roll(keys, -j, n))
            pv = None
            if vals is not None:
                pv = jnp.where(upper, _roll(vals, j, n), _roll(vals, -j, n))
            mine_first = cmp(keys, vals, pk, pv)
            first_pos = ~upper
            fwd_blk = (pos & lvl) == 0
            keep = fwd_blk == (first_pos == mine_first)
            keys = jnp.where(keep, keys, pk)
            if vals is not None:
                vals = jnp.where(keep, vals, pv)
            j //= 2
        lvl *= 2
    return keys, vals


def _cmp_desc(a, ia, b, ib):
    return (a > b) | ((a == b) & (ia < ib))


def _cmp_asc(a, ia, b, ib):
    return a < b


def _select_body(scores_ref, gpu_ref, cpu_ref):
    s = scores_ref[...].reshape(H, N_IMG)
    pad = jnp.full((H, S - N_IMG), -1.0, jnp.float32)
    s4 = jnp.concatenate([s, pad], axis=1)
    idx = lax.broadcasted_iota(jnp.int32, (H, S), 1)
    _, sidx = _bitonic(s4, idx, S, _cmp_desc)
    top = sidx[:, :TOP_K]
    rem = sidx[:, TOP_K:N_IMG]
    big = jnp.full((H, 2048 - N_REM), 1 << 30, jnp.int32)
    rem_p = jnp.concatenate([rem, big], axis=1)
    rem_s, _ = _bitonic(rem_p, None, 2048, _cmp_asc)
    rem_s = rem_s[:, :N_REM]
    hoff = lax.broadcasted_iota(jnp.int32, (H, 1), 0) * S
    text = lax.broadcasted_iota(jnp.int32, (H, N_TXT), 1) + N_IMG
    gpu_ref[...] = jnp.concatenate([text, top], axis=1) + hoff
    cpu_ref[...] = rem_s + hoff


_scores_call = pl.pallas_call(
    _scores_body,
    grid=(H, N_TBLK),
    in_specs=[
        pl.BlockSpec((1, T_BLK, D), lambda h, t: (h, N_IMG // T_BLK + t, 0)),
        pl.BlockSpec((1, N_IMG, D), lambda h, t: (h, 0, 0)),
    ],
    out_specs=pl.BlockSpec((1, 1, N_IMG), lambda h, t: (h, 0, 0)),
    out_shape=jax.ShapeDtypeStruct((H, 1, N_IMG), jnp.float32),
)

_select_call = pl.pallas_call(
    _select_body,
    out_shape=(
        jax.ShapeDtypeStruct((H, N_GPU), jnp.int32),
        jax.ShapeDtypeStruct((H, N_REM), jnp.int32),
    ),
)


@functools.partial(
    pl.kernel,
    out_type=(
        jax.ShapeDtypeStruct((H * N_GPU, D), jnp.float32),
        jax.ShapeDtypeStruct((H * N_GPU, D), jnp.float32),
        jax.ShapeDtypeStruct((H * N_REM, D), jnp.float32),
        jax.ShapeDtypeStruct((H * N_REM, D), jnp.float32),
    ),
    mesh=plsc.VectorSubcoreMesh(core_axis_name="c", subcore_axis_name="s"),
    scratch_types=[
        pltpu.VMEM((G_PER,), jnp.int32),
        pltpu.VMEM((C_PER,), jnp.int32),
        pltpu.VMEM((G_PER, D), jnp.float32),
        pltpu.SemaphoreType.DMA,
    ],
)
def _gather_call(ktab, vtab, gidx, cidx, gk, gv, ck, cv,
                 gidx_v, cidx_v, rows, sem):
    wid = lax.axis_index("s") * 2 + lax.axis_index("c")
    gb = wid * G_PER
    cb = wid * C_PER
    pltpu.sync_copy(gidx.at[pl.ds(gb, G_PER)], gidx_v)
    pltpu.sync_copy(cidx.at[pl.ds(cb, C_PER)], cidx_v)
    pltpu.async_copy(ktab.at[gidx_v], rows, sem).wait()
    pltpu.sync_copy(rows, gk.at[pl.ds(gb, G_PER)])
    pltpu.async_copy(vtab.at[gidx_v], rows, sem).wait()
    pltpu.sync_copy(rows, gv.at[pl.ds(gb, G_PER)])
    sub = rows.at[pl.ds(0, C_PER)]
    pltpu.async_copy(ktab.at[cidx_v], sub, sem).wait()
    pltpu.sync_copy(sub, ck.at[pl.ds(cb, C_PER)])
    pltpu.async_copy(vtab.at[cidx_v], sub, sem).wait()
    pltpu.sync_copy(sub, cv.at[pl.ds(cb, C_PER)])


def kernel(query_states, key_states, value_states, image_mask):
    del image_mask
    q = query_states.reshape(H, S, D)
    k = key_states.reshape(H, S, D)
    v = value_states.reshape(H, S, D)
    scores = _scores_call(q, k)
    gpu_idx, cpu_idx = _select_call(scores)
    gk, gv, ck, cv = _gather_call(
        k.reshape(H * S, D), v.reshape(H * S, D),
        gpu_idx.reshape(H * N_GPU), cpu_idx.reshape(H * N_REM))
    return (gk.reshape(1, H, N_GPU, D), gv.reshape(1, H, N_GPU, D),
            ck.reshape(1, H, N_REM, D), cv.reshape(1, H, N_REM, D))


# speedup vs baseline: 1.1703x; 1.1703x over previous
import functools

import jax
import jax.numpy as jnp
from jax import lax
from jax.experimental import pallas as pl
from jax.experimental.pallas import tpu as pltpu
from jax.experimental.pallas import tpu_sc as plsc

H = 16
S = 4096
D = 64
N_IMG = 3072
N_TXT = S - N_IMG
TOP_K = N_IMG // 2
N_REM = N_IMG - TOP_K
N_GPU = N_TXT + TOP_K

NW = 32
G_PER = H * N_GPU // NW
C_PER = H * N_REM // NW


def _attn_mass_scores(query_states, key_states, image_mask):
    im = image_mask[0]
    img_idx = jnp.nonzero(im, size=N_IMG)[0]
    image_keys = key_states[0][:, img_idx, :]
    text_queries = query_states[0][:, N_IMG:, :]
    q = jnp.transpose(text_queries[None], (0, 2, 1, 3))
    k = jnp.transpose(image_keys[None], (0, 2, 1, 3))
    d = q.shape[-1]
    logits = jnp.einsum('bthd,bnhd->bhtn', q, k) / jnp.sqrt(jnp.float32(d))
    probs = jax.nn.softmax(logits, axis=-1)
    return probs.sum(axis=2)[0]


def _roll(x, shift, n):
    return pltpu.roll(x, shift % n, 1)


def _bitonic(keys, vals, n, cmp):
    pos = lax.broadcasted_iota(jnp.int32, keys.shape, 1)
    lvl = 2
    while lvl <= n:
        j = lvl // 2
        while j >= 1:
            upper = (pos & j) != 0
            pk = jnp.where(upper, _roll(keys, j, n), _roll(keys, -j, n))
            pv = None
            if vals is not None:
                pv = jnp.where(upper, _roll(vals, j, n), _roll(vals, -j, n))
            mine_first = cmp(keys, vals, pk, pv)
            first_pos = ~upper
            fwd_blk = (pos & lvl) == 0
            keep = fwd_blk == (first_pos == mine_first)
            keys = jnp.where(keep, keys, pk)
            if vals is not None:
                vals = jnp.where(keep, vals, pv)
            j //= 2
        lvl *= 2
    return keys, vals


def _cmp_desc(a, ia, b, ib):
    return (a > b) | ((a == b) & (ia < ib))


def _cmp_asc(a, ia, b, ib):
    return a < b


def _select_body(scores_ref, gpu_ref, cpu_ref):
    s = scores_ref[...].reshape(H, N_IMG)
    pad = jnp.full((H, S - N_IMG), -1.0, jnp.float32)
    s4 = jnp.concatenate([s, pad], axis=1)
    idx = lax.broadcasted_iota(jnp.int32, (H, S), 1)
    _, sidx = _bitonic(s4, idx, S, _cmp_desc)
    top = sidx[:, :TOP_K]
    rem = sidx[:, TOP_K:N_IMG]
    big = jnp.full((H, 2048 - N_REM), 1 << 30, jnp.int32)
    rem_p = jnp.concatenate([rem, big], axis=1)
    rem_s, _ = _bitonic(rem_p, None, 2048, _cmp_asc)
    rem_s = rem_s[:, :N_REM]
    hoff = lax.broadcasted_iota(jnp.int32, (H, 1), 0) * S
    text = lax.broadcasted_iota(jnp.int32, (H, N_TXT), 1) + N_IMG
    gpu_ref[...] = jnp.concatenate([text, top], axis=1) + hoff
    cpu_ref[...] = rem_s + hoff


_select_call = pl.pallas_call(
    _select_body,
    out_shape=(
        jax.ShapeDtypeStruct((H, N_GPU), jnp.int32),
        jax.ShapeDtypeStruct((H, N_REM), jnp.int32),
    ),
)


@functools.cache
def _make_gather_call():
    return functools.partial(
        pl.kernel,
        out_type=(
            jax.ShapeDtypeStruct((H * N_GPU, D), jnp.float32),
            jax.ShapeDtypeStruct((H * N_GPU, D), jnp.float32),
            jax.ShapeDtypeStruct((H * N_REM, D), jnp.float32),
            jax.ShapeDtypeStruct((H * N_REM, D), jnp.float32),
        ),
        mesh=plsc.VectorSubcoreMesh(core_axis_name="c", subcore_axis_name="s"),
        scratch_types=[
            pltpu.VMEM((G_PER,), jnp.int32),
            pltpu.VMEM((C_PER,), jnp.int32),
            pltpu.VMEM((G_PER, D), jnp.float32),
            pltpu.SemaphoreType.DMA,
        ],
        compiler_params=pltpu.CompilerParams(use_tc_tiling_on_sc=False),
    )(_gather_body)


def _gather_body(ktab, vtab, gidx, cidx, gk, gv, ck, cv,
                 gidx_v, cidx_v, rows, sem):
    wid = lax.axis_index("s") * 2 + lax.axis_index("c")
    gb = wid * G_PER
    cb = wid * C_PER
    pltpu.sync_copy(gidx.at[pl.ds(gb, G_PER)], gidx_v)
    pltpu.sync_copy(cidx.at[pl.ds(cb, C_PER)], cidx_v)
    pltpu.async_copy(ktab.at[gidx_v], rows, sem).wait()
    pltpu.sync_copy(rows, gk.at[pl.ds(gb, G_PER)])
    pltpu.async_copy(vtab.at[gidx_v], rows, sem).wait()
    pltpu.sync_copy(rows, gv.at[pl.ds(gb, G_PER)])
    sub = rows.at[pl.ds(0, C_PER)]
    pltpu.async_copy(ktab.at[cidx_v], sub, sem).wait()
    pltpu.sync_copy(sub, ck.at[pl.ds(cb, C_PER)])
    pltpu.async_copy(vtab.at[cidx_v], sub, sem).wait()
    pltpu.sync_copy(sub, cv.at[pl.ds(cb, C_PER)])


def kernel(query_states, key_states, value_states, image_mask):
    k = key_states.reshape(H, S, D)
    v = value_states.reshape(H, S, D)
    scores = _attn_mass_scores(query_states, key_states, image_mask)
    gpu_idx, cpu_idx = _select_call(scores.reshape(H, 1, N_IMG))
    gk, gv, ck, cv = _make_gather_call()(
        k.reshape(H * S, D), v.reshape(H * S, D),
        gpu_idx.reshape(H * N_GPU), cpu_idx.reshape(H * N_REM))
    return (gk.reshape(1, H, N_GPU, D), gv.reshape(1, H, N_GPU, D),
            ck.reshape(1, H, N_REM, D), cv.reshape(1, H, N_REM, D))


# speedup vs baseline: 1.1706x; 1.0003x over previous
import functools

import jax
import jax.numpy as jnp
from jax import lax
from jax.experimental import pallas as pl
from jax.experimental.pallas import tpu as pltpu
from jax.experimental.pallas import tpu_sc as plsc

H = 16
S = 4096
D = 64
N_IMG = 3072
N_TXT = S - N_IMG
TOP_K = N_IMG // 2
N_REM = N_IMG - TOP_K
N_GPU = N_TXT + TOP_K

NW = 32
G_PER = H * N_GPU // NW
C_PER = H * N_REM // NW


def _attn_mass_scores(query_states, key_states, image_mask):
    im = image_mask[0]
    img_idx = jnp.nonzero(im, size=N_IMG)[0]
    image_keys = key_states[0][:, img_idx, :]
    text_queries = query_states[0][:, N_IMG:, :]
    q = jnp.transpose(text_queries[None], (0, 2, 1, 3))
    k = jnp.transpose(image_keys[None], (0, 2, 1, 3))
    d = q.shape[-1]
    logits = jnp.einsum('bthd,bnhd->bhtn', q, k) / jnp.sqrt(jnp.float32(d))
    probs = jax.nn.softmax(logits, axis=-1)
    return probs.sum(axis=2)[0]


def _roll(x, shift, n):
    return pltpu.roll(x, shift % n, 1)


def _bitonic(keys, vals, n, cmp):
    pos = lax.broadcasted_iota(jnp.int32, keys.shape, 1)
    lvl = 2
    while lvl <= n:
        j = lvl // 2
        while j >= 1:
            upper = (pos & j) != 0
            pk = jnp.where(upper, _roll(keys, j, n), _roll(keys, -j, n))
            pv = None
            if vals is not None:
                pv = jnp.where(upper, _roll(vals, j, n), _roll(vals, -j, n))
            mine_first = cmp(keys, vals, pk, pv)
            first_pos = ~upper
            fwd_blk = (pos & lvl) == 0
            keep = fwd_blk == (first_pos == mine_first)
            keys = jnp.where(keep, keys, pk)
            if vals is not None:
                vals = jnp.where(keep, vals, pv)
            j //= 2
        lvl *= 2
    return keys, vals


def _cmp_desc(a, ia, b, ib):
    return (a > b) | ((a == b) & (ia < ib))


def _cmp_asc(a, ia, b, ib):
    return a < b


def _select_body(scores_ref, gpu_ref, cpu_ref):
    s = scores_ref[...].reshape(H, N_IMG)
    pad = jnp.full((H, S - N_IMG), -1.0, jnp.float32)
    s4 = jnp.concatenate([s, pad], axis=1)
    idx = lax.broadcasted_iota(jnp.int32, (H, S), 1)
    _, sidx = _bitonic(s4, idx, S, _cmp_desc)
    top = sidx[:, :TOP_K]
    rem = sidx[:, TOP_K:N_IMG]
    big = jnp.full((H, 2048 - N_REM), 1 << 30, jnp.int32)
    rem_p = jnp.concatenate([rem, big], axis=1)
    rem_s, _ = _bitonic(rem_p, None, 2048, _cmp_asc)
    rem_s = rem_s[:, :N_REM]
    hoff = lax.broadcasted_iota(jnp.int32, (H, 1), 0) * S
    text = lax.broadcasted_iota(jnp.int32, (H, N_TXT), 1) + N_IMG
    gpu_ref[...] = jnp.concatenate([text, top], axis=1) + hoff
    cpu_ref[...] = rem_s + hoff


_select_call = pl.pallas_call(
    _select_body,
    out_shape=(
        jax.ShapeDtypeStruct((H, N_GPU), jnp.int32),
        jax.ShapeDtypeStruct((H, N_REM), jnp.int32),
    ),
)


@functools.cache
def _make_gather_call():
    return functools.partial(
        pl.kernel,
        out_type=(
            jax.ShapeDtypeStruct((1, H, N_GPU, D), jnp.float32),
            jax.ShapeDtypeStruct((1, H, N_GPU, D), jnp.float32),
            jax.ShapeDtypeStruct((1, H, N_REM, D), jnp.float32),
            jax.ShapeDtypeStruct((1, H, N_REM, D), jnp.float32),
        ),
        mesh=plsc.VectorSubcoreMesh(core_axis_name="c", subcore_axis_name="s"),
        scratch_types=[
            pltpu.VMEM((G_PER,), jnp.int32),
            pltpu.VMEM((C_PER,), jnp.int32),
            pltpu.VMEM((G_PER, D), jnp.float32),
            pltpu.SemaphoreType.DMA,
        ],
        compiler_params=pltpu.CompilerParams(use_tc_tiling_on_sc=False),
    )(_gather_body)


def _gather_body(ktab, vtab, gidx, cidx, gk, gv, ck, cv,
                 gidx_v, cidx_v, rows, sem):
    wid = lax.axis_index("s") * 2 + lax.axis_index("c")
    head = wid // 2
    half = wid % 2
    gb = wid * G_PER
    cb = wid * C_PER
    gh = half * G_PER
    ch = half * C_PER
    pltpu.sync_copy(gidx.at[pl.ds(gb, G_PER)], gidx_v)
    pltpu.sync_copy(cidx.at[pl.ds(cb, C_PER)], cidx_v)
    pltpu.async_copy(ktab.at[gidx_v], rows, sem).wait()
    pltpu.sync_copy(rows, gk.at[0, head, pl.ds(gh, G_PER)])
    pltpu.async_copy(vtab.at[gidx_v], rows, sem).wait()
    pltpu.sync_copy(rows, gv.at[0, head, pl.ds(gh, G_PER)])
    sub = rows.at[pl.ds(0, C_PER)]
    pltpu.async_copy(ktab.at[cidx_v], sub, sem).wait()
    pltpu.sync_copy(sub, ck.at[0, head, pl.ds(ch, C_PER)])
    pltpu.async_copy(vtab.at[cidx_v], sub, sem).wait()
    pltpu.sync_copy(sub, cv.at[0, head, pl.ds(ch, C_PER)])


def kernel(query_states, key_states, value_states, image_mask):
    k = key_states.reshape(H, S, D)
    v = value_states.reshape(H, S, D)
    scores = _attn_mass_scores(query_states, key_states, image_mask)
    gpu_idx, cpu_idx = _select_call(scores.reshape(H, 1, N_IMG))
    return _make_gather_call()(
        k.reshape(H * S, D), v.reshape(H * S, D),
        gpu_idx.reshape(H * N_GPU), cpu_idx.reshape(H * N_REM))
